# Initial kernel scaffold; baseline (speedup 1.0000x reference)
import jax
import jax.numpy as jnp
from jax.experimental import pallas as pl


def kernel(features, edge_index, W1, al1, ar1, b1, W2, al2, ar2, b2):
    raise NotImplementedError("write your pallas kernel here")


# speedup vs baseline: 22.7312x; 22.7312x over previous
import functools

import jax
import jax.numpy as jnp
from jax import lax
from jax.experimental import pallas as pl
from jax.experimental.pallas import tpu as pltpu
from jax.experimental.pallas import tpu_sc as plsc

_NC = 2
_NS = 16
_L = 16
_C = 80
_DW = 16


def _tc_in(x, W, A):
    N, _ = x.shape
    D = W.shape[1]

    def body(x_ref, w_ref, a_ref, feat_ref, e2_ref, m_ref):
        feat = jnp.dot(x_ref[...], w_ref[...],
                       preferred_element_type=jnp.float32)
        feat_ref[...] = feat
        e2 = jnp.dot(feat, a_ref[...], preferred_element_type=jnp.float32)
        e2_ref[...] = e2
        m_ref[...] = jnp.full((1, _L), jnp.maximum(2.0 * jnp.max(e2), 0.0),
                              jnp.float32)

    return pl.pallas_call(
        body,
        out_shape=(jax.ShapeDtypeStruct((N, D), jnp.float32),
                   jax.ShapeDtypeStruct((N, 2), jnp.float32),
                   jax.ShapeDtypeStruct((1, _L), jnp.float32)),
    )(x, W, A)


def _tc_mid(accp, denp, b, W, A):
    _, N, D = accp.shape

    def body(acc_ref, den_ref, b_ref, w_ref, a_ref, feat_ref, e2_ref, m_ref):
        msg = acc_ref[0] + acc_ref[1]
        den = den_ref[0][:, :1] + den_ref[1][:, :1]
        h = msg / (den + 1e-12) + b_ref[...]
        feat = jnp.dot(h, w_ref[...], preferred_element_type=jnp.float32)
        feat_ref[...] = feat
        e2 = jnp.dot(feat, a_ref[...], preferred_element_type=jnp.float32)
        e2_ref[...] = e2
        m_ref[...] = jnp.full((1, _L), jnp.maximum(2.0 * jnp.max(e2), 0.0),
                              jnp.float32)

    return pl.pallas_call(
        body,
        out_shape=(jax.ShapeDtypeStruct((N, D), jnp.float32),
                   jax.ShapeDtypeStruct((N, 2), jnp.float32),
                   jax.ShapeDtypeStruct((1, _L), jnp.float32)),
    )(accp, denp, b, W, A)


def _tc_out(accp, denp, b):
    _, N, D = accp.shape

    def body(acc_ref, den_ref, b_ref, out_ref):
        msg = acc_ref[0] + acc_ref[1]
        den = den_ref[0][:, :1] + den_ref[1][:, :1]
        out_ref[...] = msg / (den + 1e-12) + b_ref[...]

    return pl.pallas_call(
        body,
        out_shape=jax.ShapeDtypeStruct((N, D), jnp.float32),
    )(accp, denp, b)


def _sc_edge(feat, e2flat, src, dst, mvec):
    N, D = feat.shape
    E = src.shape[0]
    NW = _NC * _NS
    EW = E // NW
    NCHUNK = EW // _C
    RPS = N // _NS

    mesh = plsc.VectorSubcoreMesh(core_axis_name="c", subcore_axis_name="s",
                                  num_cores=_NC, num_subcores=_NS)

    @functools.partial(
        pl.kernel,
        out_type=(jax.ShapeDtypeStruct((_NC, N, D), jnp.float32),
                  jax.ShapeDtypeStruct((_NC, N, _DW), jnp.float32)),
        mesh=mesh,
        compiler_params=pltpu.CompilerParams(use_tc_tiling_on_sc=False,
                                             needs_layout_passes=False),
        scratch_types=[
            pltpu.VMEM((2 * N,), jnp.float32),
            pltpu.VMEM((_L,), jnp.float32),
            pltpu.VMEM((_C,), jnp.int32),
            pltpu.VMEM((1, _C), jnp.int32),
            pltpu.VMEM((_C, D), jnp.float32),
            pltpu.VMEM((_C, _DW), jnp.float32),
            pltpu.VMEM((_C,), jnp.float32),
            pltpu.VMEM_SHARED((N, D), jnp.float32),
            pltpu.VMEM_SHARED((N, _DW), jnp.float32),
            pltpu.SemaphoreType.DMA,
        ],
    )
    def k(feat_h, e2_h, src_h, dst_h, m_h, accp_h, denp_h,
          e2_v, m_v, src_v, dst_v, g_v, wrow_v, w_v, acc_sh, den_sh, sem):
        c = lax.axis_index("c")
        s = lax.axis_index("s")
        wid = c * _NS + s
        zeros = jnp.zeros((_L,), jnp.float32)

        pltpu.sync_copy(e2_h, e2_v)
        pltpu.sync_copy(m_h, m_v)
        m = m_v[...][0]

        def zrow(r, carry):
            for j in range(D // _L):
                g_v[r, pl.ds(j * _L, _L)] = zeros
            wrow_v[r, pl.ds(0, _L)] = zeros
            return carry
        lax.fori_loop(0, _C, zrow, 0)
        for kk in range(RPS // _C):
            pltpu.sync_copy(g_v, acc_sh.at[pl.ds(s * RPS + kk * _C, _C)])
            pltpu.sync_copy(wrow_v, den_sh.at[pl.ds(s * RPS + kk * _C, _C)])
        zrem = RPS % _C
        if zrem:
            zoff = s * RPS + RPS - zrem
            pltpu.sync_copy(g_v.at[pl.ds(0, zrem)],
                            acc_sh.at[pl.ds(zoff, zrem)])
            pltpu.sync_copy(wrow_v.at[pl.ds(0, zrem)],
                            den_sh.at[pl.ds(zoff, zrem)])

        lane = lax.broadcasted_iota(jnp.int32, (_L,), 0)
        lane0 = lane == 0

        plsc.subcore_barrier()

        ebase = wid * EW

        def chunk(ci, carry):
            base = ebase + ci * _C
            pltpu.sync_copy(src_h.at[pl.ds(base, _C)], src_v)
            pltpu.sync_copy(dst_h.at[pl.ds(base, _C)], dst_v.at[0])
            gat = pltpu.async_copy(feat_h.at[src_v], g_v, sem)
            for i in range(_C // _L):
                sl = pl.ds(i * _L, _L)
                sv = src_v[sl]
                dv = dst_v[0, sl]
                el = plsc.load_gather(e2_v, [sv * 2])
                er = plsc.load_gather(e2_v, [dv * 2 + 1])
                e = el + er
                e = jnp.where(e > 0, e, 0.2 * e)
                w_v[sl] = jnp.exp(e - m)
            gat.wait()

            def scale(i, carry2):
                w16 = w_v[pl.ds(i * _L, _L)]
                for rl in range(_L):
                    r = i * _L + rl
                    wr = w16[rl]
                    for j in range(D // _L):
                        slj = pl.ds(j * _L, _L)
                        g_v[r, slj] = g_v[r, slj] * wr
                    wrow_v[r, pl.ds(0, _L)] = jnp.where(lane0, wr, 0.0)
                return carry2
            lax.fori_loop(0, _C // _L, scale, 0)

            pltpu.sync_copy(g_v, acc_sh.at[dst_v.at[0]], add=True)
            pltpu.sync_copy(wrow_v, den_sh.at[dst_v.at[0]], add=True)
            return carry

        lax.fori_loop(0, NCHUNK, chunk, 0)

        plsc.subcore_barrier()

        pltpu.sync_copy(acc_sh.at[pl.ds(s * RPS, RPS)],
                        accp_h.at[c].at[pl.ds(s * RPS, RPS)])
        pltpu.sync_copy(den_sh.at[pl.ds(s * RPS, RPS)],
                        denp_h.at[c].at[pl.ds(s * RPS, RPS)])

    return k(feat, e2flat, src, dst, mvec)


def kernel(features, edge_index, W1, al1, ar1, b1, W2, al2, ar2, b2):
    src = edge_index[0]
    dst = edge_index[1]
    A1 = jnp.stack([al1, ar1], axis=1)
    A2 = jnp.stack([al2, ar2], axis=1)

    feat1, e21, m1 = _tc_in(features, W1, A1)
    accp1, denp1 = _sc_edge(feat1, e21.reshape(-1), src, dst, m1.reshape(-1))
    feat2, e22, m2 = _tc_mid(accp1, denp1, b1.reshape(1, -1), W2, A2)
    accp2, denp2 = _sc_edge(feat2, e22.reshape(-1), src, dst, m2.reshape(-1))
    out = _tc_out(accp2, denp2, b2.reshape(1, -1))
    return out


# speedup vs baseline: 43.7481x; 1.9246x over previous
import functools

import jax
import jax.numpy as jnp
from jax import lax
from jax.experimental import pallas as pl
from jax.experimental.pallas import tpu as pltpu
from jax.experimental.pallas import tpu_sc as plsc

_NC = 2
_NS = 16
_L = 16
_C = 80


def _tc_in(x, W, A):
    N, _ = x.shape
    D = W.shape[1]

    def body(x_ref, w_ref, a_ref, feat_ref, e2_ref, m_ref):
        feat = jnp.dot(x_ref[...], w_ref[...],
                       preferred_element_type=jnp.float32)
        feat_ref[...] = feat
        e2 = jnp.dot(feat, a_ref[...], preferred_element_type=jnp.float32)
        e2_ref[...] = e2
        m_ref[...] = jnp.full((1, _L), jnp.maximum(2.0 * jnp.max(e2), 0.0),
                              jnp.float32)

    return pl.pallas_call(
        body,
        out_shape=(jax.ShapeDtypeStruct((N, D), jnp.float32),
                   jax.ShapeDtypeStruct((N, 2), jnp.float32),
                   jax.ShapeDtypeStruct((1, _L), jnp.float32)),
    )(x, W, A)


def _tc_mid(accp, den_col, b, W, A):
    _, N, D = accp.shape

    def body(acc_ref, den_ref, b_ref, w_ref, a_ref, feat_ref, e2_ref, m_ref):
        msg = acc_ref[0] + acc_ref[1]
        h = msg / (den_ref[...] + 1e-12) + b_ref[...]
        feat = jnp.dot(h, w_ref[...], preferred_element_type=jnp.float32)
        feat_ref[...] = feat
        e2 = jnp.dot(feat, a_ref[...], preferred_element_type=jnp.float32)
        e2_ref[...] = e2
        m_ref[...] = jnp.full((1, _L), jnp.maximum(2.0 * jnp.max(e2), 0.0),
                              jnp.float32)

    return pl.pallas_call(
        body,
        out_shape=(jax.ShapeDtypeStruct((N, D), jnp.float32),
                   jax.ShapeDtypeStruct((N, 2), jnp.float32),
                   jax.ShapeDtypeStruct((1, _L), jnp.float32)),
    )(accp, den_col, b, W, A)


def _tc_out(accp, den_col, b):
    _, N, D = accp.shape

    def body(acc_ref, den_ref, b_ref, out_ref):
        msg = acc_ref[0] + acc_ref[1]
        out_ref[...] = msg / (den_ref[...] + 1e-12) + b_ref[...]

    return pl.pallas_call(
        body,
        out_shape=jax.ShapeDtypeStruct((N, D), jnp.float32),
    )(accp, den_col, b)


def _sc_edge(feat, e2flat, src, dst, mvec):
    N, D = feat.shape
    E = src.shape[0]
    NW = _NC * _NS
    EW = E // NW
    NCHUNK = EW // _C
    NPAIR = (NCHUNK - 1) // 2
    RPS = N // _NS

    mesh = plsc.VectorSubcoreMesh(core_axis_name="c", subcore_axis_name="s",
                                  num_cores=_NC, num_subcores=_NS)

    @functools.partial(
        pl.kernel,
        out_type=(jax.ShapeDtypeStruct((_NC, N, D), jnp.float32),
                  jax.ShapeDtypeStruct((_NC, _NS, N), jnp.float32)),
        mesh=mesh,
        compiler_params=pltpu.CompilerParams(use_tc_tiling_on_sc=False,
                                             needs_layout_passes=False),
        scratch_types=[
            pltpu.VMEM((2 * N,), jnp.float32),
            pltpu.VMEM((_L,), jnp.float32),
            pltpu.VMEM((_C,), jnp.int32),
            pltpu.VMEM((_C,), jnp.int32),
            pltpu.VMEM((1, _C), jnp.int32),
            pltpu.VMEM((1, _C), jnp.int32),
            pltpu.VMEM((_C, D), jnp.float32),
            pltpu.VMEM((_C, D), jnp.float32),
            pltpu.VMEM((N,), jnp.float32),
            pltpu.VMEM_SHARED((N, D), jnp.float32),
            pltpu.SemaphoreType.DMA,
            pltpu.SemaphoreType.DMA,
            pltpu.SemaphoreType.DMA,
        ],
    )
    def k(feat_h, e2_h, src_h, dst_h, m_h, accp_h, denp_h,
          e2_v, m_v, src_a, src_b, dst_a, dst_b, g_a, g_b, den_t, acc_sh,
          semg_a, semg_b, semi):
        c = lax.axis_index("c")
        s = lax.axis_index("s")
        wid = c * _NS + s
        zeros = jnp.zeros((_L,), jnp.float32)

        pltpu.sync_copy(e2_h, e2_v)
        pltpu.sync_copy(m_h, m_v)
        m = m_v[...][0]

        def zrow(r, carry):
            for j in range(D // _L):
                g_a[r, pl.ds(j * _L, _L)] = zeros
            return carry
        lax.fori_loop(0, _C, zrow, 0)
        for kk in range(RPS // _C):
            pltpu.sync_copy(g_a, acc_sh.at[pl.ds(s * RPS + kk * _C, _C)])
        zrem = RPS % _C
        if zrem:
            pltpu.sync_copy(g_a.at[pl.ds(0, zrem)],
                            acc_sh.at[pl.ds(s * RPS + RPS - zrem, zrem)])

        def zden(r, carry):
            den_t[pl.ds(r * _L, _L)] = zeros
            return carry
        lax.fori_loop(0, N // _L, zden, 0)

        plsc.subcore_barrier()

        ebase = wid * EW

        def issue_ids(i, sbuf, dbuf):
            base = ebase + i * _C
            pltpu.async_copy(src_h.at[pl.ds(base, _C)], sbuf, semi)
            pltpu.async_copy(dst_h.at[pl.ds(base, _C)], dbuf.at[0], semi)

        def wait_ids(i, sbuf, dbuf):
            base = ebase + i * _C
            pltpu.make_async_copy(src_h.at[pl.ds(base, _C)], sbuf,
                                  semi).wait()
            pltpu.make_async_copy(dst_h.at[pl.ds(base, _C)], dbuf.at[0],
                                  semi).wait()

        def issue_g(g, sbuf, semg):
            pltpu.async_copy(feat_h.at[sbuf], g, semg)

        def process(g, sbuf, dbuf, semg):
            pltpu.make_async_copy(feat_h.at[sbuf], g, semg).wait()

            def grp(i, carry):
                sl = pl.ds(i * _L, _L)
                sv = sbuf[sl]
                dv = dbuf[0, sl]
                el = plsc.load_gather(e2_v, [sv * 2])
                er = plsc.load_gather(e2_v, [dv * 2 + 1])
                e = el + er
                e = jnp.where(e > 0, e, 0.2 * e)
                w16 = jnp.exp(e - m)
                plsc.addupdate_scatter(den_t, [dv], w16)
                for rl in range(_L):
                    wr = w16[rl]
                    r = i * _L + rl
                    for j in range(D // _L):
                        slj = pl.ds(j * _L, _L)
                        g[r, slj] = g[r, slj] * wr
                return carry
            lax.fori_loop(0, _C // _L, grp, 0)

            pltpu.sync_copy(g, acc_sh.at[dbuf.at[0]], add=True)

        pltpu.sync_copy(src_h.at[pl.ds(ebase, _C)], src_a)
        pltpu.sync_copy(dst_h.at[pl.ds(ebase, _C)], dst_a.at[0])
        issue_g(g_a, src_a, semg_a)
        issue_ids(1, src_b, dst_b)

        def pair(kp, carry):
            i0 = 2 * kp
            i1 = 2 * kp + 1
            wait_ids(i0 + 1, src_b, dst_b)
            issue_g(g_b, src_b, semg_b)
            process(g_a, src_a, dst_a, semg_a)
            issue_ids(i0 + 2, src_a, dst_a)
            wait_ids(i1 + 1, src_a, dst_a)
            issue_g(g_a, src_a, semg_a)
            process(g_b, src_b, dst_b, semg_b)
            nxt = jnp.where(i1 + 2 >= NCHUNK, 0, i1 + 2)
            issue_ids(nxt, src_b, dst_b)
            return carry

        lax.fori_loop(0, NPAIR, pair, 0)

        process(g_a, src_a, dst_a, semg_a)
        wait_ids(0, src_b, dst_b)

        plsc.subcore_barrier()

        pltpu.sync_copy(acc_sh.at[pl.ds(s * RPS, RPS)],
                        accp_h.at[c].at[pl.ds(s * RPS, RPS)])
        pltpu.sync_copy(den_t, denp_h.at[c].at[s])

    return k(feat, e2flat, src, dst, mvec)


def kernel(features, edge_index, W1, al1, ar1, b1, W2, al2, ar2, b2):
    src = edge_index[0]
    dst = edge_index[1]
    N = features.shape[0]
    A1 = jnp.stack([al1, ar1], axis=1)
    A2 = jnp.stack([al2, ar2], axis=1)

    feat1, e21, m1 = _tc_in(features, W1, A1)
    accp1, denp1 = _sc_edge(feat1, e21.reshape(-1), src, dst, m1.reshape(-1))
    den1 = denp1.sum(axis=(0, 1)).reshape(N, 1)
    feat2, e22, m2 = _tc_mid(accp1, den1, b1.reshape(1, -1), W2, A2)
    accp2, denp2 = _sc_edge(feat2, e22.reshape(-1), src, dst, m2.reshape(-1))
    den2 = denp2.sum(axis=(0, 1)).reshape(N, 1)
    out = _tc_out(accp2, den2, b2.reshape(1, -1))
    return out


# speedup vs baseline: 52.1520x; 1.1921x over previous
import functools

import jax
import jax.numpy as jnp
from jax import lax
from jax.experimental import pallas as pl
from jax.experimental.pallas import tpu as pltpu
from jax.experimental.pallas import tpu_sc as plsc

_NC = 2
_NS = 16
_L = 16
_C = 80


def _tc_in(x, W, A):
    N, _ = x.shape
    D = W.shape[1]

    def body(x_ref, w_ref, a_ref, feat_ref, e2_ref, m_ref):
        feat = jnp.dot(x_ref[...], w_ref[...],
                       preferred_element_type=jnp.float32)
        feat_ref[...] = feat
        e2 = jnp.dot(feat, a_ref[...], preferred_element_type=jnp.float32)
        e2_ref[...] = e2
        m_ref[...] = jnp.full((1, _L), jnp.maximum(2.0 * jnp.max(e2), 0.0),
                              jnp.float32)

    return pl.pallas_call(
        body,
        out_shape=(jax.ShapeDtypeStruct((N, D), jnp.float32),
                   jax.ShapeDtypeStruct((N, 2), jnp.float32),
                   jax.ShapeDtypeStruct((1, _L), jnp.float32)),
    )(x, W, A)


def _tc_mid(accp, den_col, b, W, A):
    _, N, D = accp.shape

    def body(acc_ref, den_ref, b_ref, w_ref, a_ref, feat_ref, e2_ref, m_ref):
        msg = acc_ref[0] + acc_ref[1]
        h = msg / (den_ref[...] + 1e-12) + b_ref[...]
        feat = jnp.dot(h, w_ref[...], preferred_element_type=jnp.float32)
        feat_ref[...] = feat
        e2 = jnp.dot(feat, a_ref[...], preferred_element_type=jnp.float32)
        e2_ref[...] = e2
        m_ref[...] = jnp.full((1, _L), jnp.maximum(2.0 * jnp.max(e2), 0.0),
                              jnp.float32)

    return pl.pallas_call(
        body,
        out_shape=(jax.ShapeDtypeStruct((N, D), jnp.float32),
                   jax.ShapeDtypeStruct((N, 2), jnp.float32),
                   jax.ShapeDtypeStruct((1, _L), jnp.float32)),
    )(accp, den_col, b, W, A)


def _tc_out(accp, den_col, b):
    _, N, D = accp.shape

    def body(acc_ref, den_ref, b_ref, out_ref):
        msg = acc_ref[0] + acc_ref[1]
        out_ref[...] = msg / (den_ref[...] + 1e-12) + b_ref[...]

    return pl.pallas_call(
        body,
        out_shape=jax.ShapeDtypeStruct((N, D), jnp.float32),
    )(accp, den_col, b)


def _sc_edge(feat, e2flat, src, dst, mvec):
    N, D = feat.shape
    E = src.shape[0]
    NW = _NC * _NS
    EW = E // NW
    NCHUNK = EW // _C
    NPAIR = (NCHUNK - 1) // 2
    RPS = N // _NS

    mesh = plsc.VectorSubcoreMesh(core_axis_name="c", subcore_axis_name="s",
                                  num_cores=_NC, num_subcores=_NS)

    @functools.partial(
        pl.kernel,
        out_type=(jax.ShapeDtypeStruct((_NC, N, D), jnp.float32),
                  jax.ShapeDtypeStruct((_NC, _NS, N), jnp.float32)),
        mesh=mesh,
        compiler_params=pltpu.CompilerParams(use_tc_tiling_on_sc=False,
                                             needs_layout_passes=False),
        scratch_types=[
            pltpu.VMEM((2 * N,), jnp.float32),
            pltpu.VMEM((_L,), jnp.float32),
            pltpu.VMEM((_C,), jnp.int32),
            pltpu.VMEM((_C,), jnp.int32),
            pltpu.VMEM((1, _C), jnp.int32),
            pltpu.VMEM((1, _C), jnp.int32),
            pltpu.VMEM((1, _C), jnp.int32),
            pltpu.VMEM((1, _C), jnp.int32),
            pltpu.VMEM((_C, D), jnp.float32),
            pltpu.VMEM((_C, D), jnp.float32),
            pltpu.VMEM((N,), jnp.float32),
            pltpu.VMEM_SHARED((N, D), jnp.float32),
            pltpu.SemaphoreType.DMA,
            pltpu.SemaphoreType.DMA,
            pltpu.SemaphoreType.DMA,
            pltpu.SemaphoreType.DMA,
            pltpu.SemaphoreType.DMA,
        ],
    )
    def k(feat_h, e2_h, src_h, dst_h, m_h, accp_h, denp_h,
          e2_v, m_v, src_a, src_b, dst_a, dst_b, dstS_a, dstS_b,
          g_a, g_b, den_t, acc_sh, semg_a, semg_b, semi, semS_a, semS_b):
        c = lax.axis_index("c")
        s = lax.axis_index("s")
        wid = c * _NS + s
        zeros = jnp.zeros((_L,), jnp.float32)

        pltpu.sync_copy(e2_h, e2_v)
        pltpu.sync_copy(m_h, m_v)
        m = m_v[...][0]

        def zrow(r, carry):
            for j in range(D // _L):
                g_a[r, pl.ds(j * _L, _L)] = zeros
                g_b[r, pl.ds(j * _L, _L)] = zeros
            return carry
        lax.fori_loop(0, _C, zrow, 0)
        izeros = jnp.zeros((_L,), jnp.int32)
        for i in range(_C // _L):
            dstS_a[0, pl.ds(i * _L, _L)] = izeros
            dstS_b[0, pl.ds(i * _L, _L)] = izeros
        for kk in range(RPS // _C):
            pltpu.sync_copy(g_a, acc_sh.at[pl.ds(s * RPS + kk * _C, _C)])
        zrem = RPS % _C
        if zrem:
            pltpu.sync_copy(g_a.at[pl.ds(0, zrem)],
                            acc_sh.at[pl.ds(s * RPS + RPS - zrem, zrem)])

        def zden(r, carry):
            den_t[pl.ds(r * _L, _L)] = zeros
            return carry
        lax.fori_loop(0, N // _L, zden, 0)

        plsc.subcore_barrier()

        ebase = wid * EW

        def issue_ids(i, sbuf, dbuf):
            base = ebase + i * _C
            pltpu.async_copy(src_h.at[pl.ds(base, _C)], sbuf, semi)
            pltpu.async_copy(dst_h.at[pl.ds(base, _C)], dbuf.at[0], semi)

        def wait_ids(i, sbuf, dbuf):
            base = ebase + i * _C
            pltpu.make_async_copy(src_h.at[pl.ds(base, _C)], sbuf,
                                  semi).wait()
            pltpu.make_async_copy(dst_h.at[pl.ds(base, _C)], dbuf.at[0],
                                  semi).wait()

        def issue_g(g, sbuf, semg):
            pltpu.async_copy(feat_h.at[sbuf], g, semg)

        def wait_scat(g, dS, semS):
            pltpu.make_async_copy(g, acc_sh.at[dS.at[0]], semS).wait()

        def process(g, sbuf, dbuf, dS, semg, semS):
            pltpu.make_async_copy(feat_h.at[sbuf], g, semg).wait()

            def grp(i, carry):
                sl = pl.ds(i * _L, _L)
                sv = sbuf[sl]
                dv = dbuf[0, sl]
                dS[0, sl] = dv
                el = plsc.load_gather(e2_v, [sv * 2])
                er = plsc.load_gather(e2_v, [dv * 2 + 1])
                e = el + er
                e = jnp.where(e > 0, e, 0.2 * e)
                w16 = jnp.exp(e - m)
                plsc.addupdate_scatter(den_t, [dv], w16)
                for rl in range(_L):
                    wr = w16[rl]
                    r = i * _L + rl
                    for j in range(D // _L):
                        slj = pl.ds(j * _L, _L)
                        g[r, slj] = g[r, slj] * wr
                return carry
            lax.fori_loop(0, _C // _L, grp, 0)

            pltpu.async_copy(g, acc_sh.at[dS.at[0]], semS, add=True)

        pltpu.async_copy(g_a, acc_sh.at[dstS_a.at[0]], semS_a, add=True)
        pltpu.async_copy(g_b, acc_sh.at[dstS_b.at[0]], semS_b, add=True)

        pltpu.sync_copy(src_h.at[pl.ds(ebase, _C)], src_a)
        pltpu.sync_copy(dst_h.at[pl.ds(ebase, _C)], dst_a.at[0])
        wait_scat(g_a, dstS_a, semS_a)
        issue_g(g_a, src_a, semg_a)
        issue_ids(1, src_b, dst_b)

        def pair(kp, carry):
            i0 = 2 * kp
            i1 = 2 * kp + 1
            wait_ids(i0 + 1, src_b, dst_b)
            wait_scat(g_b, dstS_b, semS_b)
            issue_g(g_b, src_b, semg_b)
            process(g_a, src_a, dst_a, dstS_a, semg_a, semS_a)
            issue_ids(i0 + 2, src_a, dst_a)
            wait_ids(i1 + 1, src_a, dst_a)
            wait_scat(g_a, dstS_a, semS_a)
            issue_g(g_a, src_a, semg_a)
            process(g_b, src_b, dst_b, dstS_b, semg_b, semS_b)
            nxt = jnp.where(i1 + 2 >= NCHUNK, 0, i1 + 2)
            issue_ids(nxt, src_b, dst_b)
            return carry

        lax.fori_loop(0, NPAIR, pair, 0)

        process(g_a, src_a, dst_a, dstS_a, semg_a, semS_a)
        wait_ids(0, src_b, dst_b)
        wait_scat(g_a, dstS_a, semS_a)
        wait_scat(g_b, dstS_b, semS_b)

        plsc.subcore_barrier()

        pltpu.sync_copy(acc_sh.at[pl.ds(s * RPS, RPS)],
                        accp_h.at[c].at[pl.ds(s * RPS, RPS)])
        pltpu.sync_copy(den_t, denp_h.at[c].at[s])

    return k(feat, e2flat, src, dst, mvec)


def kernel(features, edge_index, W1, al1, ar1, b1, W2, al2, ar2, b2):
    src = edge_index[0]
    dst = edge_index[1]
    N = features.shape[0]
    A1 = jnp.stack([al1, ar1], axis=1)
    A2 = jnp.stack([al2, ar2], axis=1)

    feat1, e21, m1 = _tc_in(features, W1, A1)
    accp1, denp1 = _sc_edge(feat1, e21.reshape(-1), src, dst, m1.reshape(-1))
    den1 = denp1.sum(axis=(0, 1)).reshape(N, 1)
    feat2, e22, m2 = _tc_mid(accp1, den1, b1.reshape(1, -1), W2, A2)
    accp2, denp2 = _sc_edge(feat2, e22.reshape(-1), src, dst, m2.reshape(-1))
    den2 = denp2.sum(axis=(0, 1)).reshape(N, 1)
    out = _tc_out(accp2, den2, b2.reshape(1, -1))
    return out
